# baseline (speedup 1.0000x reference)
import functools

import jax
import jax.numpy as jnp
from jax import lax
from jax.experimental import pallas as pl

_ROWS_PER_STEP = 8


def _fused_body(keep_k, x_ref, pruned_ref, mask_ref):
    tokens = jnp.mean(jnp.abs(x_ref[...]), axis=1)
    tbits = lax.bitcast_convert_type(tokens, jnp.int32)
    r = tokens.shape[0]

    def bs_step(_, carry):
        lo, hi = carry
        mid = lo + ((hi - lo + 1) >> 1)
        cnt = jnp.sum(jnp.where(tbits >= mid, 1.0, 0.0), axis=1, keepdims=True)
        ge = cnt >= float(keep_k)
        return jnp.where(ge, mid, lo), jnp.where(ge, hi, mid - 1)

    lo0 = jnp.zeros((r, 1), jnp.int32)
    hi0 = jnp.full((r, 1), 0x7F800000, jnp.int32)
    thr, _ = lax.fori_loop(0, 31, bs_step, (lo0, hi0), unroll=True)

    gt = tbits > thr
    eq = tbits == thr
    need = float(keep_k) - jnp.sum(jnp.where(gt, 1.0, 0.0), axis=1, keepdims=True)

    hw = tokens.shape[1]
    col = lax.broadcasted_iota(jnp.int32, tokens.shape, 1)

    def tie_step(_, carry):
        lo, hi = carry
        mid = (lo + hi) >> 1
        cnt = jnp.sum(jnp.where(eq & (col < mid), 1.0, 0.0), axis=1, keepdims=True)
        ge = cnt >= need
        return jnp.where(ge, lo, mid), jnp.where(ge, mid, hi)

    nbits = max(1, hw.bit_length())
    tlo0 = jnp.zeros((r, 1), jnp.int32)
    thi0 = jnp.full((r, 1), hw, jnp.int32)
    _, cutoff = lax.fori_loop(0, nbits, tie_step, (tlo0, thi0), unroll=True)
    keep = gt | (eq & (col < cutoff))
    mask_ref[...] = keep.astype(jnp.int32)
    pruned_ref[...] = x_ref[...] * keep[:, None, :].astype(jnp.float32)


def kernel(x):
    B, T, C, H, W = x.shape
    BT, HW = B * T, H * W
    keep_k = max(1, int(HW * 0.5))
    R = _ROWS_PER_STEP
    x3 = x.reshape(BT, C, HW)

    pruned3, mask_i = pl.pallas_call(
        functools.partial(_fused_body, keep_k),
        grid=(BT // R,),
        in_specs=[pl.BlockSpec((R, C, HW), lambda i: (i, 0, 0))],
        out_specs=[
            pl.BlockSpec((R, C, HW), lambda i: (i, 0, 0)),
            pl.BlockSpec((R, HW), lambda i: (i, 0)),
        ],
        out_shape=[
            jax.ShapeDtypeStruct((BT, C, HW), x.dtype),
            jax.ShapeDtypeStruct((BT, HW), jnp.int32),
        ],
    )(x3)

    pruned = pruned3.reshape(B, T, C, H, W)
    mask_2d = mask_i.astype(bool).reshape(B, T, H, W)
    return (pruned, mask_2d, mask_2d)


# speedup vs baseline: 1.2915x; 1.2915x over previous
import functools

import jax
import jax.numpy as jnp
from jax import lax
from jax.experimental import pallas as pl

_ROWS_PER_STEP = 32


def _fused_body(keep_k, x_ref, pruned_ref, mask_ref):
    tokens = jnp.mean(jnp.abs(x_ref[...]), axis=1)
    tbits = lax.bitcast_convert_type(tokens, jnp.int32)
    r = tokens.shape[0]

    def bs_step(_, carry):
        lo, hi = carry
        mid = lo + ((hi - lo + 1) >> 1)
        cnt = jnp.sum(jnp.where(tbits >= mid, 1.0, 0.0), axis=1, keepdims=True)
        ge = cnt >= float(keep_k)
        return jnp.where(ge, mid, lo), jnp.where(ge, hi, mid - 1)

    lo0 = jnp.zeros((r, 1), jnp.int32)
    hi0 = jnp.full((r, 1), 0x7F800000, jnp.int32)
    thr, _ = lax.fori_loop(0, 31, bs_step, (lo0, hi0), unroll=True)

    gt = tbits > thr
    eq = tbits == thr
    need = float(keep_k) - jnp.sum(jnp.where(gt, 1.0, 0.0), axis=1, keepdims=True)

    hw = tokens.shape[1]
    col = lax.broadcasted_iota(jnp.int32, tokens.shape, 1)

    def tie_step(_, carry):
        lo, hi = carry
        mid = (lo + hi) >> 1
        cnt = jnp.sum(jnp.where(eq & (col < mid), 1.0, 0.0), axis=1, keepdims=True)
        ge = cnt >= need
        return jnp.where(ge, lo, mid), jnp.where(ge, mid, hi)

    nbits = max(1, hw.bit_length())
    tlo0 = jnp.zeros((r, 1), jnp.int32)
    thi0 = jnp.full((r, 1), hw, jnp.int32)
    _, cutoff = lax.fori_loop(0, nbits, tie_step, (tlo0, thi0), unroll=True)
    keep = gt | (eq & (col < cutoff))
    mask_ref[...] = keep.astype(jnp.int32)
    pruned_ref[...] = x_ref[...] * keep[:, None, :].astype(jnp.float32)


def kernel(x):
    B, T, C, H, W = x.shape
    BT, HW = B * T, H * W
    keep_k = max(1, int(HW * 0.5))
    R = _ROWS_PER_STEP
    x3 = x.reshape(BT, C, HW)

    pruned3, mask_i = pl.pallas_call(
        functools.partial(_fused_body, keep_k),
        grid=(BT // R,),
        in_specs=[pl.BlockSpec((R, C, HW), lambda i: (i, 0, 0))],
        out_specs=[
            pl.BlockSpec((R, C, HW), lambda i: (i, 0, 0)),
            pl.BlockSpec((R, HW), lambda i: (i, 0)),
        ],
        out_shape=[
            jax.ShapeDtypeStruct((BT, C, HW), x.dtype),
            jax.ShapeDtypeStruct((BT, HW), jnp.int32),
        ],
    )(x3)

    pruned = pruned3.reshape(B, T, C, H, W)
    mask_2d = mask_i.astype(bool).reshape(B, T, H, W)
    return (pruned, mask_2d, mask_2d)
